# baseline (speedup 1.0000x reference)
import functools

import jax
import jax.numpy as jnp
from jax import lax
from jax.experimental import pallas as pl
from jax.experimental.pallas import tpu as pltpu
from jax.experimental.pallas import tpu_sc as plsc

BATCH = 16384
D = 128
NC = 2
NS = 16
NW = NC * NS
CHUNK = 128
ROWS_PER_W = BATCH // NW
CHUNKS_PER_W = ROWS_PER_W // CHUNK


def _sc_gather(user_idx, item_idx, user_table, item_table):
    mesh = plsc.VectorSubcoreMesh(
        core_axis_name="c", subcore_axis_name="s", num_cores=NC, num_subcores=NS
    )

    @functools.partial(
        pl.kernel,
        mesh=mesh,
        out_type=(
            jax.ShapeDtypeStruct((BATCH, D), jnp.float32),
            jax.ShapeDtypeStruct((BATCH, D), jnp.float32),
        ),
        scratch_types=[
            pltpu.VMEM((ROWS_PER_W,), jnp.int32),
            pltpu.VMEM((ROWS_PER_W,), jnp.int32),
            pltpu.VMEM((4, CHUNK, D), jnp.float32),
            pltpu.SemaphoreType.DMA((4,)),
            pltpu.SemaphoreType.DMA((4,)),
        ],
    )
    def gather_kernel(ui_hbm, ii_hbm, ut_hbm, it_hbm, u_out, i_out,
                      idx_u, idx_i, bufs, gsem, wsem):
        wid = lax.axis_index("s") * NC + lax.axis_index("c")
        base = wid * CHUNKS_PER_W
        pltpu.sync_copy(ui_hbm.at[pl.ds(base * CHUNK, ROWS_PER_W)], idx_u)
        pltpu.sync_copy(ii_hbm.at[pl.ds(base * CHUNK, ROWS_PER_W)], idx_i)

        NT = 2 * CHUNKS_PER_W

        def chunk(t):
            j = t % CHUNKS_PER_W
            if t < CHUNKS_PER_W:
                return ut_hbm, idx_u.at[pl.ds(j * CHUNK, CHUNK)], u_out, j
            return it_hbm, idx_i.at[pl.ds(j * CHUNK, CHUNK)], i_out, j

        ghandles = [None] * NT
        whandles = [None] * NT
        for t in range(NT):
            m = t % 4
            if t >= 4:
                whandles[t - 4].wait()
            table, idxref, out, j = chunk(t)
            ghandles[t] = pltpu.async_copy(table.at[idxref], bufs.at[m], gsem.at[m])
            if t >= 3:
                tt = t - 3
                mm = tt % 4
                tbl2, _, out2, j2 = chunk(tt)
                ghandles[tt].wait()
                whandles[tt] = pltpu.async_copy(
                    bufs.at[mm], out2.at[pl.ds((base + j2) * CHUNK, CHUNK)], wsem.at[mm]
                )
        for tt in range(NT - 3, NT):
            mm = tt % 4
            _, _, out2, j2 = chunk(tt)
            ghandles[tt].wait()
            whandles[tt] = pltpu.async_copy(
                bufs.at[mm], out2.at[pl.ds((base + j2) * CHUNK, CHUNK)], wsem.at[mm]
            )
        for tt in range(NT - 4, NT):
            whandles[tt].wait()

    return gather_kernel(user_idx, item_idx, user_table, item_table)


def _tc_mlp(u_rows, i_rows, Wt, b2):
    BM = 1024

    def body(u_ref, i_ref, wt_ref, b_ref, o_ref):
        wt = wt_ref[...].astype(jnp.bfloat16)
        u = u_ref[...].astype(jnp.bfloat16)
        iv = i_ref[...].astype(jnp.bfloat16)
        acc = jnp.dot(u, wt[0:D, :], preferred_element_type=jnp.float32)
        acc += jnp.dot(iv, wt[D : 2 * D, :], preferred_element_type=jnp.float32)
        acc += b_ref[...]
        o_ref[...] = jnp.maximum(acc, 0.0)

    return pl.pallas_call(
        body,
        grid=(BATCH // BM,),
        in_specs=[
            pl.BlockSpec((BM, D), lambda i: (i, 0)),
            pl.BlockSpec((BM, D), lambda i: (i, 0)),
            pl.BlockSpec((2 * D, D), lambda i: (0, 0)),
            pl.BlockSpec((1, D), lambda i: (0, 0)),
        ],
        out_specs=pl.BlockSpec((BM, D), lambda i: (i, 0)),
        out_shape=jax.ShapeDtypeStruct((BATCH, D), jnp.float32),
    )(u_rows, i_rows, Wt, b2)


def kernel(user_indices, item_indices, user_table, item_table, W1, b1):
    u_rows, i_rows = _sc_gather(user_indices, item_indices, user_table, item_table)
    Wt = W1.T
    b2 = b1.reshape(1, D)
    return _tc_mlp(u_rows, i_rows, Wt, b2)


# speedup vs baseline: 1.0981x; 1.0981x over previous
import functools

import jax
import jax.numpy as jnp
from jax import lax
from jax.experimental import pallas as pl
from jax.experimental.pallas import tpu as pltpu
from jax.experimental.pallas import tpu_sc as plsc

BATCH = 16384
D = 128
NC = 2
NS = 16
NW = NC * NS
CHUNK = 128
ROWS_PER_W = BATCH // NW
CHUNKS_PER_W = ROWS_PER_W // CHUNK


def _sc_gather(user_idx, item_idx, user_table, item_table):
    mesh = plsc.VectorSubcoreMesh(
        core_axis_name="c", subcore_axis_name="s", num_cores=NC, num_subcores=NS
    )

    @functools.partial(
        pl.kernel,
        mesh=mesh,
        out_type=(
            jax.ShapeDtypeStruct((BATCH, D), jnp.float32),
            jax.ShapeDtypeStruct((BATCH, D), jnp.float32),
        ),
        scratch_types=[
            pltpu.VMEM((ROWS_PER_W,), jnp.int32),
            pltpu.VMEM((ROWS_PER_W,), jnp.int32),
            pltpu.VMEM((4, CHUNK, D), jnp.float32),
            pltpu.SemaphoreType.DMA((4,)),
            pltpu.SemaphoreType.DMA((4,)),
        ],
    )
    def gather_kernel(ui_hbm, ii_hbm, ut_hbm, it_hbm, u_out, i_out,
                      idx_u, idx_i, bufs, gsem, wsem):
        wid = lax.axis_index("s") * NC + lax.axis_index("c")
        base = wid * CHUNKS_PER_W
        pltpu.sync_copy(ui_hbm.at[pl.ds(base * CHUNK, ROWS_PER_W)], idx_u)
        pltpu.sync_copy(ii_hbm.at[pl.ds(base * CHUNK, ROWS_PER_W)], idx_i)

        NT = 2 * CHUNKS_PER_W

        def chunk(t):
            j = t % CHUNKS_PER_W
            if t < CHUNKS_PER_W:
                return ut_hbm, idx_u.at[pl.ds(j * CHUNK, CHUNK)], u_out, j
            return it_hbm, idx_i.at[pl.ds(j * CHUNK, CHUNK)], i_out, j

        ghandles = [None] * NT
        whandles = [None] * NT
        for t in range(NT):
            m = t % 4
            if t >= 4:
                whandles[t - 4].wait()
            table, idxref, out, j = chunk(t)
            ghandles[t] = pltpu.async_copy(table.at[idxref], bufs.at[m], gsem.at[m])
            if t >= 3:
                tt = t - 3
                mm = tt % 4
                tbl2, _, out2, j2 = chunk(tt)
                ghandles[tt].wait()
                whandles[tt] = pltpu.async_copy(
                    bufs.at[mm], out2.at[pl.ds((base + j2) * CHUNK, CHUNK)], wsem.at[mm]
                )
        for tt in range(NT - 3, NT):
            mm = tt % 4
            _, _, out2, j2 = chunk(tt)
            ghandles[tt].wait()
            whandles[tt] = pltpu.async_copy(
                bufs.at[mm], out2.at[pl.ds((base + j2) * CHUNK, CHUNK)], wsem.at[mm]
            )
        for tt in range(NT - 4, NT):
            whandles[tt].wait()

    return gather_kernel(user_idx, item_idx, user_table, item_table)


def _tc_mlp(u_rows, i_rows, Wt, b2):
    BM = 2048

    def body(u_ref, i_ref, wt_ref, b_ref, o_ref):
        acc = jnp.dot(u_ref[...], wt_ref[0:D, :], preferred_element_type=jnp.float32)
        acc += jnp.dot(i_ref[...], wt_ref[D : 2 * D, :], preferred_element_type=jnp.float32)
        acc += b_ref[...]
        o_ref[...] = jnp.maximum(acc, 0.0)

    return pl.pallas_call(
        body,
        grid=(BATCH // BM,),
        in_specs=[
            pl.BlockSpec((BM, D), lambda i: (i, 0)),
            pl.BlockSpec((BM, D), lambda i: (i, 0)),
            pl.BlockSpec((2 * D, D), lambda i: (0, 0)),
            pl.BlockSpec((1, D), lambda i: (0, 0)),
        ],
        out_specs=pl.BlockSpec((BM, D), lambda i: (i, 0)),
        out_shape=jax.ShapeDtypeStruct((BATCH, D), jnp.float32),
    )(u_rows, i_rows, Wt, b2)


def kernel(user_indices, item_indices, user_table, item_table, W1, b1):
    u_rows, i_rows = _sc_gather(user_indices, item_indices, user_table, item_table)
    Wt = W1.T
    b2 = b1.reshape(1, D)
    return _tc_mlp(u_rows, i_rows, Wt, b2)


# speedup vs baseline: 1.1532x; 1.0502x over previous
import functools

import jax
import jax.numpy as jnp
from jax import lax
from jax.experimental import pallas as pl
from jax.experimental.pallas import tpu as pltpu
from jax.experimental.pallas import tpu_sc as plsc

BATCH = 16384
D = 128
NC = 2
NS = 16
NW = NC * NS
CHUNK = 128
ROWS_PER_W = BATCH // NW
CHUNKS_PER_W = ROWS_PER_W // CHUNK


def _sc_gather(user_idx, item_idx, user_table, item_table):
    mesh = plsc.VectorSubcoreMesh(
        core_axis_name="c", subcore_axis_name="s", num_cores=NC, num_subcores=NS
    )

    @functools.partial(
        pl.kernel,
        mesh=mesh,
        out_type=(
            jax.ShapeDtypeStruct((BATCH, D), jnp.float32),
            jax.ShapeDtypeStruct((BATCH, D), jnp.float32),
        ),
        scratch_types=[
            pltpu.VMEM((ROWS_PER_W,), jnp.int32),
            pltpu.VMEM((ROWS_PER_W,), jnp.int32),
            pltpu.VMEM((4, CHUNK, D), jnp.float32),
            pltpu.SemaphoreType.DMA((4,)),
            pltpu.SemaphoreType.DMA((4,)),
        ],
    )
    def gather_kernel(ui_hbm, ii_hbm, ut_hbm, it_hbm, u_out, i_out,
                      idx_u, idx_i, bufs, gsem, wsem):
        wid = lax.axis_index("s") * NC + lax.axis_index("c")
        base = wid * CHUNKS_PER_W
        pltpu.sync_copy(ui_hbm.at[pl.ds(base * CHUNK, ROWS_PER_W)], idx_u)
        pltpu.sync_copy(ii_hbm.at[pl.ds(base * CHUNK, ROWS_PER_W)], idx_i)

        NT = 2 * CHUNKS_PER_W

        def chunk(t):
            j = t % CHUNKS_PER_W
            if t < CHUNKS_PER_W:
                return ut_hbm, idx_u.at[pl.ds(j * CHUNK, CHUNK)], u_out, j
            return it_hbm, idx_i.at[pl.ds(j * CHUNK, CHUNK)], i_out, j

        ghandles = [None] * NT
        whandles = [None] * NT
        for t in range(NT):
            m = t % 4
            if t >= 4:
                whandles[t - 4].wait()
            table, idxref, out, j = chunk(t)
            ghandles[t] = pltpu.async_copy(table.at[idxref], bufs.at[m], gsem.at[m])
            if t >= 3:
                tt = t - 3
                mm = tt % 4
                tbl2, _, out2, j2 = chunk(tt)
                ghandles[tt].wait()
                whandles[tt] = pltpu.async_copy(
                    bufs.at[mm], out2.at[pl.ds((base + j2) * CHUNK, CHUNK)], wsem.at[mm]
                )
        for tt in range(NT - 3, NT):
            mm = tt % 4
            _, _, out2, j2 = chunk(tt)
            ghandles[tt].wait()
            whandles[tt] = pltpu.async_copy(
                bufs.at[mm], out2.at[pl.ds((base + j2) * CHUNK, CHUNK)], wsem.at[mm]
            )
        for tt in range(NT - 4, NT):
            whandles[tt].wait()

    return gather_kernel(user_idx, item_idx, user_table, item_table)


def _tc_mlp(u_rows, i_rows, W1, b2):
    BM = 4096

    def body(u_ref, i_ref, w_ref, b_ref, o_ref):
        dn = (((1,), (1,)), ((), ()))
        acc = lax.dot_general(u_ref[...], w_ref[:, 0:D], dn,
                              preferred_element_type=jnp.float32)
        acc += lax.dot_general(i_ref[...], w_ref[:, D : 2 * D], dn,
                               preferred_element_type=jnp.float32)
        acc += b_ref[...]
        o_ref[...] = jnp.maximum(acc, 0.0)

    return pl.pallas_call(
        body,
        grid=(BATCH // BM,),
        in_specs=[
            pl.BlockSpec((BM, D), lambda i: (i, 0)),
            pl.BlockSpec((BM, D), lambda i: (i, 0)),
            pl.BlockSpec((D, 2 * D), lambda i: (0, 0)),
            pl.BlockSpec((1, D), lambda i: (0, 0)),
        ],
        out_specs=pl.BlockSpec((BM, D), lambda i: (i, 0)),
        out_shape=jax.ShapeDtypeStruct((BATCH, D), jnp.float32),
    )(u_rows, i_rows, W1, b2)


def kernel(user_indices, item_indices, user_table, item_table, W1, b1):
    u_rows, i_rows = _sc_gather(user_indices, item_indices, user_table, item_table)
    b2 = b1.reshape(1, D)
    return _tc_mlp(u_rows, i_rows, W1, b2)


# speedup vs baseline: 1.1585x; 1.0046x over previous
import functools

import jax
import jax.numpy as jnp
from jax import lax
from jax.experimental import pallas as pl
from jax.experimental.pallas import tpu as pltpu
from jax.experimental.pallas import tpu_sc as plsc

BATCH = 16384
D = 128
NC = 2
NS = 16
NW = NC * NS
CHUNK = 128
ROWS_PER_W = BATCH // NW
CHUNKS_PER_W = ROWS_PER_W // CHUNK
NB = 6


def _sc_gather(user_idx, item_idx, user_table, item_table):
    mesh = plsc.VectorSubcoreMesh(
        core_axis_name="c", subcore_axis_name="s", num_cores=NC, num_subcores=NS
    )

    @functools.partial(
        pl.kernel,
        mesh=mesh,
        out_type=(
            jax.ShapeDtypeStruct((BATCH, D), jnp.float32),
            jax.ShapeDtypeStruct((BATCH, D), jnp.float32),
        ),
        scratch_types=[
            pltpu.VMEM((ROWS_PER_W,), jnp.int32),
            pltpu.VMEM((ROWS_PER_W,), jnp.int32),
            pltpu.VMEM((NB, CHUNK, D), jnp.float32),
            pltpu.SemaphoreType.DMA((NB,)),
            pltpu.SemaphoreType.DMA((NB,)),
        ],
    )
    def gather_kernel(ui_hbm, ii_hbm, ut_hbm, it_hbm, u_out, i_out,
                      idx_u, idx_i, bufs, gsem, wsem):
        wid = lax.axis_index("s") * NC + lax.axis_index("c")
        base = wid * CHUNKS_PER_W
        pltpu.sync_copy(ui_hbm.at[pl.ds(base * CHUNK, ROWS_PER_W)], idx_u)
        pltpu.sync_copy(ii_hbm.at[pl.ds(base * CHUNK, ROWS_PER_W)], idx_i)

        NT = 2 * CHUNKS_PER_W

        def chunk(t):
            j = t % CHUNKS_PER_W
            if t < CHUNKS_PER_W:
                return ut_hbm, idx_u.at[pl.ds(j * CHUNK, CHUNK)], u_out, j
            return it_hbm, idx_i.at[pl.ds(j * CHUNK, CHUNK)], i_out, j

        GLEAD = 3
        ghandles = [None] * NT
        whandles = [None] * NT
        for t in range(NT):
            m = t % NB
            if t >= NB:
                whandles[t - NB].wait()
            table, idxref, out, j = chunk(t)
            ghandles[t] = pltpu.async_copy(table.at[idxref], bufs.at[m], gsem.at[m])
            if t >= GLEAD:
                tt = t - GLEAD
                mm = tt % NB
                _, _, out2, j2 = chunk(tt)
                ghandles[tt].wait()
                whandles[tt] = pltpu.async_copy(
                    bufs.at[mm], out2.at[pl.ds((base + j2) * CHUNK, CHUNK)], wsem.at[mm]
                )
        for tt in range(NT - GLEAD, NT):
            mm = tt % NB
            _, _, out2, j2 = chunk(tt)
            ghandles[tt].wait()
            whandles[tt] = pltpu.async_copy(
                bufs.at[mm], out2.at[pl.ds((base + j2) * CHUNK, CHUNK)], wsem.at[mm]
            )
        for tt in range(max(0, NT - NB), NT):
            whandles[tt].wait()

    return gather_kernel(user_idx, item_idx, user_table, item_table)


def _tc_mlp(u_rows, i_rows, W1, b2):
    BM = 4096

    def body(u_ref, i_ref, w_ref, b_ref, o_ref):
        dn = (((1,), (1,)), ((), ()))
        acc = lax.dot_general(u_ref[...], w_ref[:, 0:D], dn,
                              preferred_element_type=jnp.float32)
        acc += lax.dot_general(i_ref[...], w_ref[:, D : 2 * D], dn,
                               preferred_element_type=jnp.float32)
        acc += b_ref[...]
        o_ref[...] = jnp.maximum(acc, 0.0)

    return pl.pallas_call(
        body,
        grid=(BATCH // BM,),
        in_specs=[
            pl.BlockSpec((BM, D), lambda i: (i, 0)),
            pl.BlockSpec((BM, D), lambda i: (i, 0)),
            pl.BlockSpec((D, 2 * D), lambda i: (0, 0)),
            pl.BlockSpec((1, D), lambda i: (0, 0)),
        ],
        out_specs=pl.BlockSpec((BM, D), lambda i: (i, 0)),
        out_shape=jax.ShapeDtypeStruct((BATCH, D), jnp.float32),
    )(u_rows, i_rows, W1, b2)


def kernel(user_indices, item_indices, user_table, item_table, W1, b1):
    u_rows, i_rows = _sc_gather(user_indices, item_indices, user_table, item_table)
    b2 = b1.reshape(1, D)
    return _tc_mlp(u_rows, i_rows, W1, b2)
